# baseline (speedup 1.0000x reference)
import jax
import jax.numpy as jnp
from jax import lax
from jax.experimental import pallas as pl
from jax.experimental.pallas import tpu as pltpu
from jax.experimental.pallas import tpu_sc as plsc

_NC = 2
_NS = 16
_NW = _NC * _NS
_TR = 8
_CW = 1024


def _sc_body(x_hbm, pe_hbm, o_hbm, peb, xb, psem, lsem, ssem):
    Bn, L, D = x_hbm.shape
    rows = L // _NW
    ntr = rows // _TR
    ncq = D // _CW
    wid = lax.axis_index("s") * _NC + lax.axis_index("c")
    rbase = wid * rows

    def issue_loads(r0, c0, slot):
        pltpu.async_copy(
            pe_hbm.at[pl.ds(r0, _TR), pl.ds(c0, _CW)], peb.at[slot],
            psem.at[slot])
        pltpu.async_copy(
            x_hbm.at[pl.ds(0, Bn), pl.ds(r0, _TR), pl.ds(c0, _CW)],
            xb.at[slot], lsem.at[slot])

    def wait_pe(slot):
        pltpu.make_async_copy(
            pe_hbm.at[pl.ds(0, _TR), pl.ds(0, _CW)], peb.at[slot],
            psem.at[slot]).wait()

    def wait_x(slot):
        pltpu.make_async_copy(
            x_hbm.at[pl.ds(0, Bn), pl.ds(0, _TR), pl.ds(0, _CW)],
            xb.at[slot], lsem.at[slot]).wait()

    def drain_stores(slot):
        pltpu.make_async_copy(
            xb.at[slot],
            o_hbm.at[pl.ds(0, Bn), pl.ds(0, _TR), pl.ds(0, _CW)],
            ssem.at[slot]).wait()

    issue_loads(rbase, 0, 0)

    def tr_body(tr, _):
        r0 = rbase + tr * _TR
        for cq in range(ncq):
            s = cq & 1
            c0 = cq * _CW
            if cq < ncq - 1:
                if cq == 0:
                    @pl.when(tr >= 1)
                    def _():
                        drain_stores(1 - s)
                else:
                    drain_stores(1 - s)
                issue_loads(r0, c0 + _CW, 1 - s)
            else:
                @pl.when(tr + 1 < ntr)
                def _():
                    drain_stores(1 - s)
                    issue_loads(r0 + _TR, 0, 1 - s)
            wait_pe(s)
            wait_x(s)

            pes = peb.at[s]
            for r in range(_TR):
                @plsc.parallel_loop(0, _CW, step=16, unroll=8)
                def _(c):
                    sl = pl.ds(c, 16)
                    pv = pes[r, sl]
                    for b in range(Bn):
                        plsc.addupdate(xb.at[s, b, r, sl], pv)

            pltpu.async_copy(
                xb.at[s],
                o_hbm.at[pl.ds(0, Bn), pl.ds(r0, _TR), pl.ds(c0, _CW)],
                ssem.at[s])
        return 0

    lax.fori_loop(0, ntr, tr_body, 0)

    drain_stores(0)
    drain_stores(1)


def kernel(x, pe):
    B, L, D = x.shape
    return pl.kernel(
        _sc_body,
        out_type=jax.ShapeDtypeStruct((B, L, D), x.dtype),
        mesh=plsc.VectorSubcoreMesh(core_axis_name="c", subcore_axis_name="s"),
        scratch_types=[
            pltpu.VMEM((2, _TR, _CW), jnp.float32),
            pltpu.VMEM((2, B, _TR, _CW), jnp.float32),
            pltpu.SemaphoreType.DMA((2,)),
            pltpu.SemaphoreType.DMA((2,)),
            pltpu.SemaphoreType.DMA((2,)),
        ],
        compiler_params=pltpu.CompilerParams(use_tc_tiling_on_sc=True),
    )(x, pe)


# speedup vs baseline: 1.0086x; 1.0086x over previous
import jax
import jax.numpy as jnp
from jax import lax
from jax.experimental import pallas as pl
from jax.experimental.pallas import tpu as pltpu
from jax.experimental.pallas import tpu_sc as plsc

_NC = 2
_NS = 16
_NW = _NC * _NS
_TR = 8
_CW = 1024


def _sc_body(x_hbm, pe_hbm, o_hbm, peb, xb, psem, lsem, ssem):
    Bn, L, D = x_hbm.shape
    rows = L // _NW
    ntr = rows // _TR
    ncq = D // _CW
    wid = lax.axis_index("s") * _NC + lax.axis_index("c")
    rbase = wid * rows

    def issue_loads(r0, c0, slot):
        pltpu.async_copy(
            pe_hbm.at[pl.ds(r0, _TR), pl.ds(c0, _CW)], peb.at[slot],
            psem.at[slot])
        pltpu.async_copy(
            x_hbm.at[pl.ds(0, Bn), pl.ds(r0, _TR), pl.ds(c0, _CW)],
            xb.at[slot], lsem.at[slot])

    def wait_pe(slot):
        pltpu.make_async_copy(
            pe_hbm.at[pl.ds(0, _TR), pl.ds(0, _CW)], peb.at[slot],
            psem.at[slot]).wait()

    def wait_x(slot):
        pltpu.make_async_copy(
            x_hbm.at[pl.ds(0, Bn), pl.ds(0, _TR), pl.ds(0, _CW)],
            xb.at[slot], lsem.at[slot]).wait()

    def drain_stores(slot):
        pltpu.make_async_copy(
            xb.at[slot],
            o_hbm.at[pl.ds(0, Bn), pl.ds(0, _TR), pl.ds(0, _CW)],
            ssem.at[slot]).wait()

    issue_loads(rbase, 0, 0)

    def tr_body(tr, _):
        r0 = rbase + tr * _TR
        for cq in range(ncq):
            s = cq & 1
            c0 = cq * _CW
            if cq < ncq - 1:
                if cq == 0:
                    @pl.when(tr >= 1)
                    def _():
                        drain_stores(1 - s)
                else:
                    drain_stores(1 - s)
                issue_loads(r0, c0 + _CW, 1 - s)
            else:
                @pl.when(tr + 1 < ntr)
                def _():
                    drain_stores(1 - s)
                    issue_loads(r0 + _TR, 0, 1 - s)
            wait_pe(s)
            wait_x(s)

            pes = peb.at[s]
            for r in range(_TR):
                @plsc.parallel_loop(0, _CW, step=16, unroll=4)
                def _(c):
                    sl = pl.ds(c, 16)
                    pv = pes[r, sl]
                    for b in range(Bn):
                        plsc.addupdate(xb.at[s, b, r, sl], pv)

            pltpu.async_copy(
                xb.at[s],
                o_hbm.at[pl.ds(0, Bn), pl.ds(r0, _TR), pl.ds(c0, _CW)],
                ssem.at[s])
        return 0

    lax.fori_loop(0, ntr, tr_body, 0)

    drain_stores(0)
    drain_stores(1)


def kernel(x, pe):
    B, L, D = x.shape
    return pl.kernel(
        _sc_body,
        out_type=jax.ShapeDtypeStruct((B, L, D), x.dtype),
        mesh=plsc.VectorSubcoreMesh(core_axis_name="c", subcore_axis_name="s"),
        scratch_types=[
            pltpu.VMEM((2, _TR, _CW), jnp.float32),
            pltpu.VMEM((2, B, _TR, _CW), jnp.float32),
            pltpu.SemaphoreType.DMA((2,)),
            pltpu.SemaphoreType.DMA((2,)),
            pltpu.SemaphoreType.DMA((2,)),
        ],
        compiler_params=pltpu.CompilerParams(use_tc_tiling_on_sc=True),
    )(x, pe)
